# Initial kernel scaffold; baseline (speedup 1.0000x reference)
import jax
import jax.numpy as jnp
from jax.experimental import pallas as pl


def kernel(feat0, feat1, feat2, params):
    raise NotImplementedError("write your pallas kernel here")


# speedup vs baseline: 3.2570x; 3.2570x over previous
import functools

import jax
import jax.numpy as jnp
from jax.experimental import pallas as pl
from jax.experimental.pallas import tpu as pltpu

_EPS = 1e-5


def _dwconv(y, taps, okh, okw, w):
    acc = y * taps[:, 4:5]
    for dh in (-1, 0, 1):
        for dw in (-1, 0, 1):
            if dh == 0 and dw == 0:
                continue
            idx = (dh + 1) * 3 + (dw + 1)
            r = jnp.roll(y, -(dh * w + dw), axis=1)
            if dh != 0 and dw != 0:
                m = jnp.logical_and(okh[dh], okw[dw])
            elif dh != 0:
                m = okh[dh]
            else:
                m = okw[dw]
            acc = acc + jnp.where(m, r * taps[:, idx : idx + 1], 0.0)
    return acc


def _body(x_ref, w1, g1, b1, k1, gd1, bd1, a2, b2w, g2, bt2, k2, gd2, bd2,
          pa, pb, bp, cls_ref, obj_ref, reg_ref, s1, s2, xbuf, dsem, *, h, w):
    n = x_ref.shape[0]
    hw = h * w
    inv = 1.0 / float(n * hw)

    def x_copy(i):
        return pltpu.make_async_copy(x_ref.at[i], xbuf.at[i % 2],
                                     dsem.at[i % 2])

    pos = jax.lax.broadcasted_iota(jnp.int32, (1, hw), 1)
    hpos = pos // w
    wpos = pos % w
    okh = {-1: hpos >= 1, 1: hpos <= h - 2}
    okw = {-1: wpos >= 1, 1: wpos <= w - 2}

    def bn_scale(s, q, g, b):
        m = s * inv
        v = q * inv - m * m
        sc = g[:, 0] * jax.lax.rsqrt(v + _EPS)
        return sc[:, None], (b[:, 0] - m * sc)[:, None]

    s = jnp.zeros((96,), jnp.float32)
    q = jnp.zeros((96,), jnp.float32)
    x_copy(0).start()
    for i in range(n):
        x_copy(i).wait()
        if i + 1 < n:
            x_copy(i + 1).start()
        z = jnp.dot(w1[...], xbuf[i % 2], preferred_element_type=jnp.float32)
        s1[i] = z
        s = s + jnp.sum(z, axis=1)
        q = q + jnp.sum(z * z, axis=1)
    sc, of = bn_scale(s, q, g1[...], b1[...])

    s = jnp.zeros((96,), jnp.float32)
    q = jnp.zeros((96,), jnp.float32)
    for i in range(n):
        y1 = jnp.clip(s1[i] * sc + of, 0.0, 6.0)
        s1[i] = y1
        d = _dwconv(y1, k1[...], okh, okw, w)
        s2[i] = d
        s = s + jnp.sum(d, axis=1)
        q = q + jnp.sum(d * d, axis=1)
    sc, of = bn_scale(s, q, gd1[...], bd1[...])

    s = jnp.zeros((96,), jnp.float32)
    q = jnp.zeros((96,), jnp.float32)
    for i in range(n):
        y2 = jnp.clip(s2[i] * sc + of, 0.0, 6.0)
        z2 = (jnp.dot(a2[...], s1[i], preferred_element_type=jnp.float32)
              + jnp.dot(b2w[...], y2, preferred_element_type=jnp.float32))
        s2[i] = z2
        s = s + jnp.sum(z2, axis=1)
        q = q + jnp.sum(z2 * z2, axis=1)
    sc, of = bn_scale(s, q, g2[...], bt2[...])

    s = jnp.zeros((96,), jnp.float32)
    q = jnp.zeros((96,), jnp.float32)
    for i in range(n):
        y3 = jnp.clip(s2[i] * sc + of, 0.0, 6.0)
        s2[i] = y3
        d2 = _dwconv(y3, k2[...], okh, okw, w)
        s1[i] = d2
        s = s + jnp.sum(d2, axis=1)
        q = q + jnp.sum(d2 * d2, axis=1)
    sc, of = bn_scale(s, q, gd2[...], bd2[...])

    for i in range(n):
        y4 = jnp.clip(s1[i] * sc + of, 0.0, 6.0)
        p = (jnp.dot(pa[...], s2[i], preferred_element_type=jnp.float32)
             + jnp.dot(pb[...], y4, preferred_element_type=jnp.float32)
             + bp[...])
        cls_ref[i] = p[:80]
        obj_ref[i] = p[80:81]
        reg_ref[i] = p[81:113]


def _stack2(pc, pr, key):
    return jnp.concatenate([pc[key], pr[key]], axis=0)


def _level(x, h, w, w1, g1, b1, k1, gd1, bd1, a2, b2w, g2, bt2, k2, gd2, bd2,
           pa, pb, bp):
    nb = x.shape[0]
    hw = h * w
    xf = x.reshape(nb, 96, hw)
    f32 = jnp.float32
    outs = pl.pallas_call(
        functools.partial(_body, h=h, w=w),
        in_specs=[pl.BlockSpec(memory_space=pl.ANY)]
        + [pl.BlockSpec(memory_space=pltpu.VMEM)] * 16,
        out_shape=[
            jax.ShapeDtypeStruct((nb, 80, hw), f32),
            jax.ShapeDtypeStruct((nb, 1, hw), f32),
            jax.ShapeDtypeStruct((nb, 32, hw), f32),
        ],
        scratch_shapes=[
            pltpu.VMEM((nb, 96, hw), f32),
            pltpu.VMEM((nb, 96, hw), f32),
            pltpu.VMEM((2, 96, hw), f32),
            pltpu.SemaphoreType.DMA((2,)),
        ],
    )(xf, w1, g1, b1, k1, gd1, bd1, a2, b2w, g2, bt2, k2, gd2, bd2, pa, pb, bp)
    cls, obj, reg = outs
    return (cls.reshape(nb, 80, h, w), obj.reshape(nb, 1, h, w),
            reg.reshape(nb, 32, h, w))


def kernel(feat0, feat1, feat2, params):
    pc1, pc2 = params['cls_conv']
    pr1, pr2 = params['reg_conv']

    w1 = _stack2(pc1, pr1, 'pw').reshape(96, 96)
    g1 = _stack2(pc1, pr1, 'pg').reshape(96, 1)
    b1 = _stack2(pc1, pr1, 'pb').reshape(96, 1)
    k1 = _stack2(pc1, pr1, 'cw').reshape(96, 9)
    gd1 = _stack2(pc1, pr1, 'cg').reshape(96, 1)
    bd1 = _stack2(pc1, pr1, 'cb').reshape(96, 1)

    wc2 = pc2['pw'].reshape(48, 96)
    wr2 = pr2['pw'].reshape(48, 96)
    z48 = jnp.zeros((48, 48), jnp.float32)
    a2 = jnp.block([[wc2[:, :48], z48], [z48, wr2[:, :48]]])
    b2w = jnp.block([[wc2[:, 48:], z48], [z48, wr2[:, 48:]]])
    g2 = _stack2(pc2, pr2, 'pg').reshape(96, 1)
    bt2 = _stack2(pc2, pr2, 'pb').reshape(96, 1)
    k2 = _stack2(pc2, pr2, 'cw').reshape(96, 9)
    gd2 = _stack2(pc2, pr2, 'cg').reshape(96, 1)
    bd2 = _stack2(pc2, pr2, 'cb').reshape(96, 1)

    shared = (w1, g1, b1, k1, gd1, bd1, a2, b2w, g2, bt2, k2, gd2, bd2)

    cls_ls, obj_ls, reg_ls = [], [], []
    for i, (f, h, w) in enumerate([(feat0, 64, 64), (feat1, 32, 32),
                                   (feat2, 16, 16)]):
        wco = jnp.concatenate([params['cls_pred'][i]['w'].reshape(80, 96),
                               params['obj_pred'][i]['w'].reshape(1, 96)],
                              axis=0)
        wrg = params['reg_pred'][i]['w'].reshape(32, 96)
        za = jnp.zeros((81, 48), jnp.float32)
        zb = jnp.zeros((32, 48), jnp.float32)
        pa = jnp.block([[wco[:, :48], za], [zb, wrg[:, :48]]])
        pb = jnp.block([[wco[:, 48:], za], [zb, wrg[:, 48:]]])
        bp = jnp.concatenate([params['cls_pred'][i]['b'],
                              params['obj_pred'][i]['b'],
                              params['reg_pred'][i]['b']]).reshape(113, 1)
        cls, obj, reg = _level(f, h, w, *shared, pa, pb, bp)
        cls_ls.append(cls)
        obj_ls.append(obj)
        reg_ls.append(reg)
    return tuple(cls_ls + obj_ls + reg_ls)


# speedup vs baseline: 4.7897x; 1.4706x over previous
import functools

import jax
import jax.numpy as jnp
from jax.experimental import pallas as pl
from jax.experimental.pallas import tpu as pltpu

_EPS = 1e-5


def _dwconv(y, taps, okh, okw, w):
    um = jnp.where(okw[-1], jnp.roll(y, 1, axis=1), 0.0)
    up = jnp.where(okw[1], jnp.roll(y, -1, axis=1), 0.0)

    def trow(dh):
        b = (dh + 1) * 3
        return (um * taps[:, b:b + 1] + y * taps[:, b + 1:b + 2]
                + up * taps[:, b + 2:b + 3])

    acc = trow(0)
    acc = acc + jnp.where(okh[-1], jnp.roll(trow(-1), w, axis=1), 0.0)
    acc = acc + jnp.where(okh[1], jnp.roll(trow(1), -w, axis=1), 0.0)
    return acc


def _body(x_ref, w1, g1, b1, k1, gd1, bd1, w2s, g2, bt2, k2, gd2, bd2,
          pas, bp, cls_ref, obj_ref, reg_ref, s1, s2, dsem, *, h, w, nchunk):
    n = x_ref.shape[0]
    hw = h * w
    total = n * hw
    cl = total // nchunk
    spc = cl // hw
    inv = 1.0 / float(total)

    pos = jax.lax.broadcasted_iota(jnp.int32, (1, cl), 1)
    wp = pos % w
    hp = (pos // w) % h
    okw = {-1: wp >= 1, 1: wp <= w - 2}
    okh = {-1: hp >= 1, 1: hp <= h - 2}

    def sl(c):
        return (slice(None), pl.ds(c * cl, cl))

    def bn_scale(s, q, g, b):
        m = s * inv
        v = q * inv - m * m
        sc = g[:, 0] * jax.lax.rsqrt(v + _EPS)
        return sc[:, None], (b[:, 0] - m * sc)[:, None]

    def stats(z, s, q):
        return s + jnp.sum(z, axis=1), q + jnp.sum(z * z, axis=1)

    copies = [pltpu.make_async_copy(x_ref.at[i],
                                    s2.at[:, pl.ds(i * hw, hw)],
                                    dsem.at[i]) for i in range(n)]
    for cp in copies:
        cp.start()

    s = jnp.zeros((96,), jnp.float32)
    q = jnp.zeros((96,), jnp.float32)
    for c in range(nchunk):
        for i in range(c * spc, (c + 1) * spc):
            copies[i].wait()
        z = jnp.dot(w1[...], s2[sl(c)].astype(jnp.bfloat16),
                    preferred_element_type=jnp.float32)
        s1[sl(c)] = z
        s, q = stats(z, s, q)
    sc, of = bn_scale(s, q, g1[...], b1[...])

    s = jnp.zeros((96,), jnp.float32)
    q = jnp.zeros((96,), jnp.float32)
    for c in range(nchunk):
        y1 = jnp.clip(s1[sl(c)] * sc + of, 0.0, 6.0)
        s1[sl(c)] = y1
        d = _dwconv(y1, k1[...], okh, okw, w)
        s2[sl(c)] = d
        s, q = stats(d, s, q)
    sc, of = bn_scale(s, q, gd1[...], bd1[...])

    s = jnp.zeros((96,), jnp.float32)
    q = jnp.zeros((96,), jnp.float32)
    for c in range(nchunk):
        y2 = jnp.clip(s2[sl(c)] * sc + of, 0.0, 6.0)
        ycat = jnp.concatenate([s1[sl(c)], y2], axis=0).astype(jnp.bfloat16)
        z2 = jnp.dot(w2s[...], ycat, preferred_element_type=jnp.float32)
        s2[sl(c)] = z2
        s, q = stats(z2, s, q)
    sc, of = bn_scale(s, q, g2[...], bt2[...])

    s = jnp.zeros((96,), jnp.float32)
    q = jnp.zeros((96,), jnp.float32)
    for c in range(nchunk):
        y3 = jnp.clip(s2[sl(c)] * sc + of, 0.0, 6.0)
        s2[sl(c)] = y3
        d2 = _dwconv(y3, k2[...], okh, okw, w)
        s1[sl(c)] = d2
        s, q = stats(d2, s, q)
    sc, of = bn_scale(s, q, gd2[...], bd2[...])

    for c in range(nchunk):
        y4 = jnp.clip(s1[sl(c)] * sc + of, 0.0, 6.0)
        ycat = jnp.concatenate([s2[sl(c)], y4], axis=0).astype(jnp.bfloat16)
        p = jnp.dot(pas[...], ycat, preferred_element_type=jnp.float32) + bp[...]
        for j in range(spc):
            i = c * spc + j
            cls_ref[i] = p[0:80, j * hw:(j + 1) * hw]
            obj_ref[i] = p[80:81, j * hw:(j + 1) * hw]
            reg_ref[i] = p[81:113, j * hw:(j + 1) * hw]


def _stack2(pc, pr, key):
    return jnp.concatenate([pc[key], pr[key]], axis=0)


def _level(x, h, w, w1, g1, b1, k1, gd1, bd1, w2s, g2, bt2, k2, gd2, bd2,
           pas, bp):
    nb = x.shape[0]
    hw = h * w
    xf = x.reshape(nb, 96, hw)
    f32 = jnp.float32
    nchunk = max(1, (nb * hw) // 8192)
    outs = pl.pallas_call(
        functools.partial(_body, h=h, w=w, nchunk=nchunk),
        in_specs=[pl.BlockSpec(memory_space=pl.ANY)]
        + [pl.BlockSpec(memory_space=pltpu.VMEM)] * 14,
        out_shape=[
            jax.ShapeDtypeStruct((nb, 80, hw), f32),
            jax.ShapeDtypeStruct((nb, 1, hw), f32),
            jax.ShapeDtypeStruct((nb, 32, hw), f32),
        ],
        scratch_shapes=[
            pltpu.VMEM((96, nb * hw), f32),
            pltpu.VMEM((96, nb * hw), f32),
            pltpu.SemaphoreType.DMA((nb,)),
        ],
    )(xf, w1, g1, b1, k1, gd1, bd1, w2s, g2, bt2, k2, gd2, bd2, pas, bp)
    cls, obj, reg = outs
    return (cls.reshape(nb, 80, h, w), obj.reshape(nb, 1, h, w),
            reg.reshape(nb, 32, h, w))


def kernel(feat0, feat1, feat2, params):
    pc1, pc2 = params['cls_conv']
    pr1, pr2 = params['reg_conv']

    w1 = _stack2(pc1, pr1, 'pw').reshape(96, 96).astype(jnp.bfloat16)
    g1 = _stack2(pc1, pr1, 'pg').reshape(96, 1)
    b1 = _stack2(pc1, pr1, 'pb').reshape(96, 1)
    k1 = _stack2(pc1, pr1, 'cw').reshape(96, 9)
    gd1 = _stack2(pc1, pr1, 'cg').reshape(96, 1)
    bd1 = _stack2(pc1, pr1, 'cb').reshape(96, 1)

    wc2 = pc2['pw'].reshape(48, 96)
    wr2 = pr2['pw'].reshape(48, 96)
    z48 = jnp.zeros((48, 48), jnp.float32)
    w2s = jnp.block([[wc2[:, :48], z48, wc2[:, 48:], z48],
                     [z48, wr2[:, :48], z48, wr2[:, 48:]]]).astype(jnp.bfloat16)
    g2 = _stack2(pc2, pr2, 'pg').reshape(96, 1)
    bt2 = _stack2(pc2, pr2, 'pb').reshape(96, 1)
    k2 = _stack2(pc2, pr2, 'cw').reshape(96, 9)
    gd2 = _stack2(pc2, pr2, 'cg').reshape(96, 1)
    bd2 = _stack2(pc2, pr2, 'cb').reshape(96, 1)

    shared = (w1, g1, b1, k1, gd1, bd1, w2s, g2, bt2, k2, gd2, bd2)

    cls_ls, obj_ls, reg_ls = [], [], []
    for i, (f, h, w) in enumerate([(feat0, 64, 64), (feat1, 32, 32),
                                   (feat2, 16, 16)]):
        wco = jnp.concatenate([params['cls_pred'][i]['w'].reshape(80, 96),
                               params['obj_pred'][i]['w'].reshape(1, 96)],
                              axis=0)
        wrg = params['reg_pred'][i]['w'].reshape(32, 96)
        za = jnp.zeros((81, 48), jnp.float32)
        zb = jnp.zeros((32, 48), jnp.float32)
        pas = jnp.block([[wco[:, :48], za, wco[:, 48:], za],
                         [zb, wrg[:, :48], zb,
                          wrg[:, 48:]]]).astype(jnp.bfloat16)
        bp = jnp.concatenate([params['cls_pred'][i]['b'],
                              params['obj_pred'][i]['b'],
                              params['reg_pred'][i]['b']]).reshape(113, 1)
        cls, obj, reg = _level(f, h, w, *shared, pas, bp)
        cls_ls.append(cls)
        obj_ls.append(obj)
        reg_ls.append(reg)
    return tuple(cls_ls + obj_ls + reg_ls)
